# baseline (speedup 1.0000x reference)
import jax
import jax.numpy as jnp
from jax import lax
from jax.experimental import pallas as pl
from jax.experimental.pallas import tpu as pltpu
from jax.experimental.pallas import tpu_sc as plsc

N_CLS = 1000
D0, D1 = 4096, 20
NC, NS, L = 2, 16, 16
NW = NC * NS
NW_ACT = 16
SLAB = D0 // NW_ACT
CSEG = 200
NSEG = N_CLS // CSEG
N_CHUNKS = D1 * NSEG


def _sc_onehot(xt_hbm, z_hbm, out_hbm, xv, buf0, buf1, sem0, sem1):
    wid = lax.axis_index("s") * NC + lax.axis_index("c")
    d0_0 = wid * SLAB
    bufs = (buf0, buf1)
    sems = (sem0, sem1)

    @pl.when(wid < NW_ACT)
    def _body():
        _work(xt_hbm, z_hbm, out_hbm, xv, buf0, buf1, sem0, sem1, d0_0, bufs, sems)


def _work(xt_hbm, z_hbm, out_hbm, xv, buf0, buf1, sem0, sem1, d0_0, bufs, sems):

    h_x = pltpu.async_copy(xt_hbm.at[:, pl.ds(d0_0, SLAB)], xv, sem0)
    h_z0 = pltpu.async_copy(z_hbm, buf0, sem1)
    h_z1 = pltpu.async_copy(z_hbm, buf1, sem1)
    h_x.wait()
    h_z0.wait()
    h_z1.wait()

    iota = lax.iota(jnp.int32, L)
    zero16 = jnp.zeros((L,), jnp.int32)
    ones = jnp.full((L,), 1.0, jnp.float32)
    zeros = jnp.zeros((L,), jnp.float32)

    def _flats(k):
        d1, s = k // NSEG, k % NSEG
        out = []
        for j in range(SLAB // L):
            cols = xv[d1, pl.ds(j * L, L)]
            cl = cols - s * CSEG
            mask = (cl >= 0) & (cl < CSEG)
            out.append((cl, iota + j * L, mask))
        return out

    def _dst(k):
        d1, s = k // NSEG, k % NSEG
        return out_hbm.at[
            pl.ds(d1, 1), pl.ds(s * CSEG, CSEG), pl.ds(d0_0, SLAB)
        ]

    def _outer(o, _):
        for b in range(2):
            k = o * 2 + b
            buf = bufs[b]

            @pl.when(o > 0)
            def _drain():
                pltpu.make_async_copy(buf, _dst(k - 2), sems[b]).wait()
                for cl, d0l, mask in _flats(k - 2):
                    plsc.store_scatter(buf, [zero16, cl, d0l], zeros, mask=mask)

            for cl, d0l, mask in _flats(k):
                plsc.store_scatter(buf, [zero16, cl, d0l], ones, mask=mask)
            pltpu.async_copy(buf, _dst(k), sems[b])
        return 0

    lax.fori_loop(0, N_CHUNKS // 2, _outer, 0)
    for b in range(2):
        pltpu.make_async_copy(bufs[b], _dst(N_CHUNKS - 2 + b), sems[b]).wait()


def kernel(x):
    xt = jnp.transpose(x.astype(jnp.int32))
    zblk = jnp.zeros((1, CSEG, SLAB), jnp.float32)
    mesh = plsc.VectorSubcoreMesh(core_axis_name="c", subcore_axis_name="s")
    out = pl.kernel(
        _sc_onehot,
        out_type=jax.ShapeDtypeStruct((D1, N_CLS, D0), jnp.float32),
        mesh=mesh,
        scratch_types=[
            pltpu.VMEM((D1, SLAB), jnp.int32),
            pltpu.VMEM((1, CSEG, SLAB), jnp.float32),
            pltpu.VMEM((1, CSEG, SLAB), jnp.float32),
            pltpu.SemaphoreType.DMA,
            pltpu.SemaphoreType.DMA,
        ],
        compiler_params=pltpu.CompilerParams(needs_layout_passes=False),
    )(xt, zblk)
    return jnp.transpose(out, (2, 0, 1))


# speedup vs baseline: 1.4744x; 1.4744x over previous
import jax
import jax.numpy as jnp
from jax import lax
from jax.experimental import pallas as pl
from jax.experimental.pallas import tpu as pltpu
from jax.experimental.pallas import tpu_sc as plsc

N_CLS = 1000
D0, D1 = 4096, 20
NC, NS, L = 2, 16, 16
NW = NC * NS
SLAB = D0 // NW
CSEG = 200
NSEG = N_CLS // CSEG
N_CHUNKS = D1 * NSEG


def _sc_onehot(xt_hbm, z_hbm, out_hbm, xv, buf0, buf1, buf2, buf3,
               sem0, sem1, sem2, sem3):
    wid = lax.axis_index("s") * NC + lax.axis_index("c")
    d0_0 = wid * SLAB
    bufs = (buf0, buf1, buf2, buf3)
    sems = (sem0, sem1, sem2, sem3)

    h_x = pltpu.async_copy(xt_hbm.at[:, pl.ds(d0_0, SLAB)], xv, sem0)
    h_z0 = pltpu.async_copy(z_hbm, buf0, sem1)
    h_z1 = pltpu.async_copy(z_hbm, buf1, sem1)
    h_z2 = pltpu.async_copy(z_hbm, buf2, sem2)
    h_z3 = pltpu.async_copy(z_hbm, buf3, sem3)
    h_x.wait()
    h_z0.wait()
    h_z1.wait()
    h_z2.wait()
    h_z3.wait()

    iota = lax.iota(jnp.int32, L)
    zero16 = jnp.zeros((L,), jnp.int32)
    ones = jnp.full((L,), 1.0, jnp.float32)
    zeros = jnp.zeros((L,), jnp.float32)

    def _flats(k):
        d1, s = k // NSEG, k % NSEG
        out = []
        for j in range(SLAB // L):
            cols = xv[d1, pl.ds(j * L, L)]
            cl = cols - s * CSEG
            mask = (cl >= 0) & (cl < CSEG)
            out.append((cl, iota + j * L, mask))
        return out

    def _dst(k):
        d1, s = k // NSEG, k % NSEG
        return out_hbm.at[
            pl.ds(d1, 1), pl.ds(s * CSEG, CSEG), pl.ds(d0_0, SLAB)
        ]

    NB = 4
    assert N_CHUNKS % NB == 0

    def _outer(o, _):
        for b in range(NB):
            k = o * NB + b
            buf = bufs[b]

            @pl.when(o > 0)
            def _drain():
                pltpu.make_async_copy(buf, _dst(k - NB), sems[b]).wait()
                for cl, d0l, mask in _flats(k - NB):
                    plsc.store_scatter(buf, [zero16, cl, d0l], zeros, mask=mask)

            for cl, d0l, mask in _flats(k):
                plsc.store_scatter(buf, [zero16, cl, d0l], ones, mask=mask)
            pltpu.async_copy(buf, _dst(k), sems[b])
        return 0

    lax.fori_loop(0, N_CHUNKS // NB, _outer, 0)
    for b in range(NB):
        pltpu.make_async_copy(bufs[b], _dst(N_CHUNKS - NB + b), sems[b]).wait()


def kernel(x):
    xt = jnp.transpose(x.astype(jnp.int32))
    zblk = jnp.zeros((1, CSEG, SLAB), jnp.float32)
    mesh = plsc.VectorSubcoreMesh(core_axis_name="c", subcore_axis_name="s")
    out = pl.kernel(
        _sc_onehot,
        out_type=jax.ShapeDtypeStruct((D1, N_CLS, D0), jnp.float32),
        mesh=mesh,
        scratch_types=[
            pltpu.VMEM((D1, SLAB), jnp.int32),
            pltpu.VMEM((1, CSEG, SLAB), jnp.float32),
            pltpu.VMEM((1, CSEG, SLAB), jnp.float32),
            pltpu.VMEM((1, CSEG, SLAB), jnp.float32),
            pltpu.VMEM((1, CSEG, SLAB), jnp.float32),
            pltpu.SemaphoreType.DMA,
            pltpu.SemaphoreType.DMA,
            pltpu.SemaphoreType.DMA,
            pltpu.SemaphoreType.DMA,
        ],
        compiler_params=pltpu.CompilerParams(needs_layout_passes=False),
    )(xt, zblk)
    return jnp.transpose(out, (2, 0, 1))


# speedup vs baseline: 1.6060x; 1.0892x over previous
import jax
import jax.numpy as jnp
from jax import lax
from jax.experimental import pallas as pl
from jax.experimental.pallas import tpu as pltpu
from jax.experimental.pallas import tpu_sc as plsc

N_CLS = 1000
D0, D1 = 4096, 20
NC, NS, L = 2, 16, 16
NW = NC * NS
SLAB = D0 // NW
CSEG = 200
NSEG = N_CLS // CSEG
N_CHUNKS = D1 * NSEG


def _sc_onehot(xt_hbm, z_hbm, out_hbm, xv, buf0, buf1, sem0, sem1):
    wid = lax.axis_index("s") * NC + lax.axis_index("c")
    d0_0 = wid * SLAB
    bufs = (buf0, buf1)
    sems = (sem0, sem1)

    h_x = pltpu.async_copy(xt_hbm.at[:, pl.ds(d0_0, SLAB)], xv, sem0)
    h_z0 = pltpu.async_copy(z_hbm, buf0, sem1)
    h_z1 = pltpu.async_copy(z_hbm, buf1, sem1)
    h_x.wait()
    h_z0.wait()
    h_z1.wait()

    iota = lax.iota(jnp.int32, L)
    zero16 = jnp.zeros((L,), jnp.int32)
    ones = jnp.full((L,), 1.0, jnp.float32)
    zeros = jnp.zeros((L,), jnp.float32)

    def _flats(k):
        d1, s = k // NSEG, k % NSEG
        out = []
        for j in range(SLAB // L):
            cols = xv[d1, pl.ds(j * L, L)]
            cl = cols - s * CSEG
            mask = (cl >= 0) & (cl < CSEG)
            out.append((cl, iota + j * L, mask))
        return out

    def _dst(k):
        d1, s = k // NSEG, k % NSEG
        return out_hbm.at[
            pl.ds(d1, 1), pl.ds(s * CSEG, CSEG), pl.ds(d0_0, SLAB)
        ]

    def _outer(o, _):
        for b in range(2):
            k = o * 2 + b
            buf = bufs[b]

            @pl.when(o > 0)
            def _drain():
                pltpu.make_async_copy(buf, _dst(k - 2), sems[b]).wait()
                for cl, d0l, mask in _flats(k - 2):
                    plsc.store_scatter(buf, [zero16, cl, d0l], zeros, mask=mask)

            for cl, d0l, mask in _flats(k):
                plsc.store_scatter(buf, [zero16, cl, d0l], ones, mask=mask)
            pltpu.async_copy(buf, _dst(k), sems[b])
        return 0

    lax.fori_loop(0, N_CHUNKS // 2, _outer, 0)
    for b in range(2):
        pltpu.make_async_copy(bufs[b], _dst(N_CHUNKS - 2 + b), sems[b]).wait()


def kernel(x):
    xt = jnp.transpose(x.astype(jnp.int32))
    zblk = jnp.zeros((1, CSEG, SLAB), jnp.float32)
    mesh = plsc.VectorSubcoreMesh(core_axis_name="c", subcore_axis_name="s")
    out = pl.kernel(
        _sc_onehot,
        out_type=jax.ShapeDtypeStruct((D1, N_CLS, D0), jnp.float32),
        mesh=mesh,
        scratch_types=[
            pltpu.VMEM((D1, SLAB), jnp.int32),
            pltpu.VMEM((1, CSEG, SLAB), jnp.float32),
            pltpu.VMEM((1, CSEG, SLAB), jnp.float32),
            pltpu.SemaphoreType.DMA,
            pltpu.SemaphoreType.DMA,
        ],
        compiler_params=pltpu.CompilerParams(needs_layout_passes=False),
    )(xt, zblk)
    return jnp.transpose(out, (2, 0, 1))
